# Initial kernel scaffold; baseline (speedup 1.0000x reference)
import jax
import jax.numpy as jnp
from jax.experimental import pallas as pl


def kernel(input_, weight):
    raise NotImplementedError("write your pallas kernel here")


# speedup vs baseline: 1.8613x; 1.8613x over previous
import functools

import jax
import jax.numpy as jnp
from jax import lax
from jax.experimental import pallas as pl
from jax.experimental.pallas import tpu as pltpu
from jax.experimental.pallas import tpu_sc as plsc

_NUM_CORES = 2
_NUM_SUBCORES = 16
_NUM_WORKERS = _NUM_CORES * _NUM_SUBCORES


@functools.partial(jax.jit, static_argnums=(2, 3, 4))
def _gather(idx, table, B, D, CH):
    b_per_w = B // _NUM_WORKERS
    n_ch = b_per_w // CH
    mesh = plsc.VectorSubcoreMesh(core_axis_name="c", subcore_axis_name="s")

    @functools.partial(
        pl.kernel,
        mesh=mesh,
        out_type=jax.ShapeDtypeStruct((B, D), jnp.float32),
        compiler_params=pltpu.CompilerParams(use_tc_tiling_on_sc=False),
        scratch_types=[
            pltpu.VMEM((b_per_w,), jnp.int32),
            pltpu.VMEM((CH, D), jnp.float32),
            pltpu.SemaphoreType.DMA,
        ],
    )
    def k(idx_hbm, table_hbm, out_hbm, idx_v, rows_v, sem):
        wid = lax.axis_index("s") * _NUM_CORES + lax.axis_index("c")
        base = wid * b_per_w
        pltpu.sync_copy(idx_hbm.at[pl.ds(base, b_per_w)], idx_v)

        def body(i, carry):
            pltpu.async_copy(
                table_hbm.at[idx_v.at[pl.ds(i * CH, CH)]], rows_v, sem
            ).wait()
            pltpu.sync_copy(rows_v, out_hbm.at[pl.ds(base + i * CH, CH)])
            return carry

        lax.fori_loop(0, n_ch, body, 0)

    return k(idx, table)


def kernel(input_, weight):
    B0, S = input_.shape
    _, D = weight.shape
    B = B0 * S
    idx = input_.reshape(B).astype(jnp.int32)
    out = _gather(idx, weight, B, D, 1024)
    return out.reshape(B0, S, D)


# speedup vs baseline: 1.8752x; 1.0075x over previous
import functools

import jax
import jax.numpy as jnp
from jax import lax
from jax.experimental import pallas as pl
from jax.experimental.pallas import tpu as pltpu
from jax.experimental.pallas import tpu_sc as plsc

_NUM_CORES = 2
_NUM_SUBCORES = 16
_NUM_WORKERS = _NUM_CORES * _NUM_SUBCORES


@functools.partial(jax.jit, static_argnums=(2, 3, 4))
def _gather(idx, table, B, D, CH):
    b_per_w = B // _NUM_WORKERS
    n_ch = b_per_w // CH
    assert n_ch >= 2 and n_ch * CH == b_per_w
    mesh = plsc.VectorSubcoreMesh(core_axis_name="c", subcore_axis_name="s")

    @functools.partial(
        pl.kernel,
        mesh=mesh,
        out_type=jax.ShapeDtypeStruct((B, D), jnp.float32),
        compiler_params=pltpu.CompilerParams(use_tc_tiling_on_sc=False),
        scratch_types=[
            pltpu.VMEM((b_per_w,), jnp.int32),
            pltpu.VMEM((CH, D), jnp.float32),
            pltpu.VMEM((CH, D), jnp.float32),
            pltpu.SemaphoreType.DMA,
            pltpu.SemaphoreType.DMA,
            pltpu.SemaphoreType.DMA,
            pltpu.SemaphoreType.DMA,
        ],
    )
    def k(idx_hbm, table_hbm, out_hbm, idx_v, rows0, rows1, g0, g1, s0, s1):
        wid = lax.axis_index("s") * _NUM_CORES + lax.axis_index("c")
        base = wid * b_per_w
        pltpu.sync_copy(idx_hbm.at[pl.ds(base, b_per_w)], idx_v)

        rows = (rows0, rows1)
        gsem = (g0, g1)
        ssem = (s0, s1)

        def start_g(i, b):
            return pltpu.async_copy(
                table_hbm.at[idx_v.at[pl.ds(i * CH, CH)]], rows[b], gsem[b]
            )

        def start_s(i, b):
            return pltpu.async_copy(
                rows[b], out_hbm.at[pl.ds(base + i * CH, CH)], ssem[b]
            )

        def wait_g(b):
            pltpu.make_async_copy(
                table_hbm.at[pl.ds(0, CH)], rows[b], gsem[b]
            ).wait()

        def wait_s(i, b):
            pltpu.make_async_copy(
                rows[b], out_hbm.at[pl.ds(base + i * CH, CH)], ssem[b]
            ).wait()

        start_g(0, 0)
        start_g(1, 1)
        wait_g(0)
        start_s(0, 0)

        T = n_ch // 2

        def round_body(t, carry):
            i0 = 2 * t
            wait_s(i0 - 2, 0)
            start_g(i0, 0)
            wait_g(1)
            start_s(i0 - 1, 1)
            wait_s(i0 - 1, 1)
            start_g(i0 + 1, 1)
            wait_g(0)
            start_s(i0, 0)
            return carry

        lax.fori_loop(1, T, round_body, 0)
        wait_g(1)
        start_s(n_ch - 1, 1)
        wait_s(n_ch - 2, 0)
        wait_s(n_ch - 1, 1)

    return k(idx, table)


def kernel(input_, weight):
    B0, S = input_.shape
    _, D = weight.shape
    B = B0 * S
    idx = input_.reshape(B).astype(jnp.int32)
    out = _gather(idx, weight, B, D, 800)
    return out.reshape(B0, S, D)
